# baseline (speedup 1.0000x reference)
import jax
import jax.numpy as jnp
from jax import lax
from jax.experimental import pallas as pl
from jax.experimental.pallas import tpu as pltpu
from jax.experimental.pallas import tpu_sc as plsc

N = 10000
E = 320000
F = 128
NC = 2
NS = 16
NW = NC * NS
N2 = N + 8
WIN = 128
NWIN = 80
EPW = WIN * NWIN
E2 = EPW * NW
NBUF = 2
NGRP = NWIN // NBUF
RPS = 624


def _sc_mesh():
    return plsc.VectorSubcoreMesh(core_axis_name="c", subcore_axis_name="s",
                                  num_cores=NC, num_subcores=NS)


def _rows_copy(sid, src, dst, nrows):
    r0 = sid * RPS
    tail = nrows - RPS * NS
    pltpu.sync_copy(src.at[pl.ds(r0, RPS)], dst.at[pl.ds(r0, RPS)])

    @pl.when(sid == 0)
    def _():
        pltpu.sync_copy(src.at[pl.ds(RPS * NS, tail)],
                        dst.at[pl.ds(RPS * NS, tail)])


def _sc_agg_body(h_hbm, src_hbm, dst_hbm, zf_hbm, acc_out,
                 dst_all, i0, i1, r0, r1,
                 is0, is1, g0, g1, s0, s1, acc_sh):
    ibuf = [i0, i1]
    rows = [r0, r1]
    isem = [is0, is1]
    gsem = [g0, g1]
    ssem = [s0, s1]
    cid = lax.axis_index("c")
    sid = lax.axis_index("s")
    wid = cid * NS + sid
    e0 = wid * EPW

    _rows_copy(sid, zf_hbm, acc_sh, N2)
    pltpu.sync_copy(dst_hbm.at[wid], dst_all)

    def idesc(j, b):
        return pltpu.make_async_copy(src_hbm.at[pl.ds(e0 + j * WIN, WIN)],
                                     ibuf[b], isem[b])

    def gdesc(j, b):
        return pltpu.make_async_copy(h_hbm.at[ibuf[b]], rows[b], gsem[b])

    def sdesc(j, b):
        return pltpu.make_async_copy(rows[b], acc_sh.at[dst_all.at[j]],
                                     ssem[b])

    idesc(0, 0).start()
    idesc(1, 1).start()
    plsc.subcore_barrier()
    idesc(0, 0).wait()
    gdesc(0, 0).start()

    def phase(g, b):
        o = 1 - b
        gdesc(g, b).wait()
        sdesc(g, b).start(add=True)

        @pl.when(g + 2 < NWIN)
        def _():
            idesc(g + 2, b).start()

        @pl.when(g >= 1)
        def _():
            sdesc(g - 1, o).wait()

        @pl.when(g + 1 < NWIN)
        def _():
            idesc(g + 1, o).wait()
            gdesc(g + 1, o).start()

    @pl.loop(0, NWIN // 2)
    def _(t):
        phase(2 * t, 0)
        phase(2 * t + 1, 1)

    sdesc(NWIN - 1, 1).wait()
    plsc.subcore_barrier()
    _rows_copy(sid, acc_sh, acc_out.at[cid], N)


def _make_sc_agg():
    return pl.kernel(
        _sc_agg_body,
        out_type=jax.ShapeDtypeStruct((NC, N, F), jnp.float32),
        mesh=_sc_mesh(),
        scratch_types=([pltpu.VMEM((NWIN, WIN), jnp.int32)]
                       + [pltpu.VMEM((WIN,), jnp.int32)] * 2
                       + [pltpu.VMEM((WIN, F), jnp.float32)] * 2
                       + [pltpu.SemaphoreType.DMA] * 6
                       + [pltpu.VMEM_SHARED((N2, F), jnp.float32)]))


def _sc_cnt_body(dst_hbm, zf_hbm, ones_hbm, cnt_out,
                 dst_all, ones_v, cnt_sh, csem):
    cid = lax.axis_index("c")
    sid = lax.axis_index("s")
    wid = cid * NS + sid

    _rows_copy(sid, zf_hbm, cnt_sh, N2)
    pltpu.sync_copy(dst_hbm.at[wid], dst_all)
    pltpu.sync_copy(ones_hbm, ones_v)
    plsc.subcore_barrier()

    @pl.loop(0, NWIN // 4)
    def _(t):
        ds = [pltpu.make_async_copy(ones_v, cnt_sh.at[dst_all.at[4 * t + b]],
                                    csem) for b in range(4)]
        for d in ds:
            d.start(add=True)
        for d in ds:
            d.wait()

    plsc.subcore_barrier()
    _rows_copy(sid, cnt_sh, cnt_out.at[cid], N)


def _make_sc_cnt():
    return pl.kernel(
        _sc_cnt_body,
        out_type=jax.ShapeDtypeStruct((NC, N, F), jnp.float32),
        mesh=_sc_mesh(),
        scratch_types=[pltpu.VMEM((NWIN, WIN), jnp.int32),
                       pltpu.VMEM((WIN, F), jnp.float32),
                       pltpu.VMEM_SHARED((N2, F), jnp.float32),
                       pltpu.SemaphoreType.DMA])


_RB = 1000
_GRID = N // _RB


def _dot(a, b):
    return jnp.dot(a, b, preferred_element_type=jnp.float32,
                   precision=lax.Precision.HIGHEST)


def _in_proj_body(x_ref, w_ref, b_ref, o_ref):
    o_ref[...] = _dot(x_ref[...], w_ref[...]) + b_ref[...]


def _in_proj(x, w, b):
    return pl.pallas_call(
        _in_proj_body,
        grid=(_GRID,),
        in_specs=[pl.BlockSpec((_RB, F), lambda i: (i, 0)),
                  pl.BlockSpec((F, F), lambda i: (0, 0)),
                  pl.BlockSpec((1, F), lambda i: (0, 0))],
        out_specs=pl.BlockSpec((_RB, F), lambda i: (i, 0)),
        out_shape=jax.ShapeDtypeStruct((N, F), jnp.float32),
    )(x, w, b.reshape(1, F))


def _mix_body(p_ref, c_ref, h_ref, wl_ref, wr_ref, b_ref, y_ref, s_ref):
    cnt = c_ref[0, :, 0:1] + c_ref[1, :, 0:1]
    inv = 1.0 / jnp.maximum(cnt, 1.0)
    agg = (p_ref[0] + p_ref[1]) * inv
    y = _dot(agg, wl_ref[...]) + _dot(h_ref[...], wr_ref[...]) + b_ref[...]
    y_ref[...] = y
    stats = jnp.concatenate([jnp.sum(y, axis=0, keepdims=True),
                             jnp.sum(y * y, axis=0, keepdims=True)], axis=0)
    i = pl.program_id(0)

    @pl.when(i == 0)
    def _():
        s_ref[...] = stats

    @pl.when(i > 0)
    def _():
        s_ref[...] += stats


def _mix(parts, cnts, h, wl, wr, b):
    return pl.pallas_call(
        _mix_body,
        grid=(_GRID,),
        in_specs=[pl.BlockSpec((NC, _RB, F), lambda i: (0, i, 0)),
                  pl.BlockSpec((NC, _RB, F), lambda i: (0, i, 0)),
                  pl.BlockSpec((_RB, F), lambda i: (i, 0)),
                  pl.BlockSpec((F, F), lambda i: (0, 0)),
                  pl.BlockSpec((F, F), lambda i: (0, 0)),
                  pl.BlockSpec((1, F), lambda i: (0, 0))],
        out_specs=[pl.BlockSpec((_RB, F), lambda i: (i, 0)),
                   pl.BlockSpec((2, F), lambda i: (0, 0))],
        out_shape=[jax.ShapeDtypeStruct((N, F), jnp.float32),
                   jax.ShapeDtypeStruct((2, F), jnp.float32)],
    )(parts, cnts, h, wl, wr, b.reshape(1, F))


def _normed(y, s_ref, w_ref, b_ref, a_ref):
    mean = s_ref[0:1, :] * (1.0 / N)
    msq = s_ref[1:2, :] * (1.0 / N)
    a = a_ref[...]
    var = msq - mean * mean * (2.0 * a - a * a)
    xc = y - a * mean
    t = w_ref[...] * xc / jnp.sqrt(var + 1e-5) + b_ref[...]
    return jnp.maximum(t, 0.1 * t)


def _norm_body(y_ref, s_ref, w_ref, b_ref, a_ref, o_ref):
    o_ref[...] = _normed(y_ref[...], s_ref, w_ref, b_ref, a_ref)


def _norm(y, s, w, b, a):
    return pl.pallas_call(
        _norm_body,
        grid=(_GRID,),
        in_specs=[pl.BlockSpec((_RB, F), lambda i: (i, 0)),
                  pl.BlockSpec((2, F), lambda i: (0, 0)),
                  pl.BlockSpec((1, F), lambda i: (0, 0)),
                  pl.BlockSpec((1, F), lambda i: (0, 0)),
                  pl.BlockSpec((1, F), lambda i: (0, 0))],
        out_specs=pl.BlockSpec((_RB, F), lambda i: (i, 0)),
        out_shape=jax.ShapeDtypeStruct((N, F), jnp.float32),
    )(y, s, w.reshape(1, F), b.reshape(1, F), a.reshape(1, F))


def _norm_out_body(y_ref, s_ref, w_ref, b_ref, a_ref, wo_ref, bo_ref, o_ref):
    t = _normed(y_ref[...], s_ref, w_ref, b_ref, a_ref)
    o_ref[...] = _dot(t, wo_ref[...]) + bo_ref[...]


def _norm_out(y, s, w, b, a, wo, bo):
    return pl.pallas_call(
        _norm_out_body,
        grid=(_GRID,),
        in_specs=[pl.BlockSpec((_RB, F), lambda i: (i, 0)),
                  pl.BlockSpec((2, F), lambda i: (0, 0)),
                  pl.BlockSpec((1, F), lambda i: (0, 0)),
                  pl.BlockSpec((1, F), lambda i: (0, 0)),
                  pl.BlockSpec((1, F), lambda i: (0, 0)),
                  pl.BlockSpec((F, F), lambda i: (0, 0)),
                  pl.BlockSpec((1, F), lambda i: (0, 0))],
        out_specs=pl.BlockSpec((_RB, F), lambda i: (i, 0)),
        out_shape=jax.ShapeDtypeStruct((N, F), jnp.float32),
    )(y, s, w.reshape(1, F), b.reshape(1, F), a.reshape(1, F),
      wo, bo.reshape(1, F))


def kernel(x, edge_index, W_in, b_in,
           W1_l, b1_l, W1_r, gn1_w, gn1_b, gn1_a,
           W2_l, b2_l, W2_r, gn2_w, gn2_b, gn2_a,
           W3_l, b3_l, W3_r, gn3_w, gn3_b, gn3_a,
           W_out, b_out):
    npad = E2 - E
    src = jnp.concatenate(
        [edge_index[0], jnp.zeros((npad,), jnp.int32)])
    dst = jnp.concatenate(
        [edge_index[1], jnp.full((npad,), N, jnp.int32)]).reshape(NW, NWIN, WIN)
    zf = jnp.zeros((N2, F), jnp.float32)
    ones = jnp.ones((WIN, F), jnp.float32)

    _sc_agg = _make_sc_agg()
    _sc_cnt = _make_sc_cnt()

    cnt = _sc_cnt(dst, zf, ones)
    h0 = _in_proj(x, W_in, b_in)
    p1 = _sc_agg(h0, src, dst, zf)
    y1, s1 = _mix(p1, cnt, h0, W1_l, W1_r, b1_l)
    h1 = _norm(y1, s1, gn1_w, gn1_b, gn1_a)
    p2 = _sc_agg(h1, src, dst, zf)
    y2, s2 = _mix(p2, cnt, h1, W2_l, W2_r, b2_l)
    h2 = _norm(y2, s2, gn2_w, gn2_b, gn2_a)
    p3 = _sc_agg(h2, src, dst, zf)
    y3, s3 = _mix(p3, cnt, h2, W3_l, W3_r, b3_l)
    return _norm_out(y3, s3, gn3_w, gn3_b, gn3_a, W_out, b_out)


# speedup vs baseline: 1.0011x; 1.0011x over previous
import jax
import jax.numpy as jnp
from jax import lax
from jax.experimental import pallas as pl
from jax.experimental.pallas import tpu as pltpu
from jax.experimental.pallas import tpu_sc as plsc

N = 10000
E = 320000
F = 128
NC = 2
NS = 16
NW = NC * NS
N2 = N + 8
WIN = 128
NWIN = 80
EPW = WIN * NWIN
E2 = EPW * NW
NBUF = 2
NGRP = NWIN // NBUF
RPS = 624


def _sc_mesh():
    return plsc.VectorSubcoreMesh(core_axis_name="c", subcore_axis_name="s",
                                  num_cores=NC, num_subcores=NS)


def _rows_copy(sid, src, dst, nrows):
    r0 = sid * RPS
    tail = nrows - RPS * NS
    pltpu.sync_copy(src.at[pl.ds(r0, RPS)], dst.at[pl.ds(r0, RPS)])

    @pl.when(sid == 0)
    def _():
        pltpu.sync_copy(src.at[pl.ds(RPS * NS, tail)],
                        dst.at[pl.ds(RPS * NS, tail)])


def _sc_agg_body(h_hbm, src_hbm, dst_hbm, zf_hbm, acc_out,
                 dst_all, i0, i1, r0, r1,
                 is0, is1, g0, g1, s0, s1, acc_sh):
    ibuf = [i0, i1]
    rows = [r0, r1]
    isem = [is0, is1]
    gsem = [g0, g1]
    ssem = [s0, s1]
    cid = lax.axis_index("c")
    sid = lax.axis_index("s")
    wid = cid * NS + sid
    e0 = wid * EPW

    _rows_copy(sid, zf_hbm, acc_sh, N2)
    pltpu.sync_copy(dst_hbm.at[wid], dst_all)

    def idesc(j, b):
        return pltpu.make_async_copy(src_hbm.at[pl.ds(e0 + j * WIN, WIN)],
                                     ibuf[b], isem[b])

    def gdesc(j, b):
        return pltpu.make_async_copy(h_hbm.at[ibuf[b]], rows[b], gsem[b])

    def sdesc(j, b):
        return pltpu.make_async_copy(rows[b], acc_sh.at[dst_all.at[j]],
                                     ssem[b])

    idesc(0, 0).start()
    idesc(1, 1).start()
    plsc.subcore_barrier()
    idesc(0, 0).wait()
    gdesc(0, 0).start()

    def phase(g, b):
        o = 1 - b
        gdesc(g, b).wait()

        @pl.when(g >= 1)
        def _():
            sdesc(g - 1, o).wait()

        sdesc(g, b).start(add=True)

        @pl.when(g + 2 < NWIN)
        def _():
            idesc(g + 2, b).start()

        @pl.when(g + 1 < NWIN)
        def _():
            idesc(g + 1, o).wait()
            gdesc(g + 1, o).start()

    @pl.loop(0, NWIN // 2)
    def _(t):
        phase(2 * t, 0)
        phase(2 * t + 1, 1)

    sdesc(NWIN - 1, 1).wait()
    plsc.subcore_barrier()
    _rows_copy(sid, acc_sh, acc_out.at[cid], N)


def _make_sc_agg():
    return pl.kernel(
        _sc_agg_body,
        out_type=jax.ShapeDtypeStruct((NC, N, F), jnp.float32),
        mesh=_sc_mesh(),
        scratch_types=([pltpu.VMEM((NWIN, WIN), jnp.int32)]
                       + [pltpu.VMEM((WIN,), jnp.int32)] * 2
                       + [pltpu.VMEM((WIN, F), jnp.float32)] * 2
                       + [pltpu.SemaphoreType.DMA] * 6
                       + [pltpu.VMEM_SHARED((N2, F), jnp.float32)]))


def _sc_cnt_body(dst_hbm, zf_hbm, ones_hbm, cnt_out,
                 dst_all, ones_v, cnt_sh, csem):
    cid = lax.axis_index("c")
    sid = lax.axis_index("s")
    wid = cid * NS + sid

    _rows_copy(sid, zf_hbm, cnt_sh, N2)
    pltpu.sync_copy(dst_hbm.at[wid], dst_all)
    pltpu.sync_copy(ones_hbm, ones_v)
    plsc.subcore_barrier()

    @pl.loop(0, NWIN // 4)
    def _(t):
        ds = [pltpu.make_async_copy(ones_v, cnt_sh.at[dst_all.at[4 * t + b]],
                                    csem) for b in range(4)]
        for d in ds:
            d.start(add=True)
        for d in ds:
            d.wait()

    plsc.subcore_barrier()
    _rows_copy(sid, cnt_sh, cnt_out.at[cid], N)


def _make_sc_cnt():
    return pl.kernel(
        _sc_cnt_body,
        out_type=jax.ShapeDtypeStruct((NC, N, F), jnp.float32),
        mesh=_sc_mesh(),
        scratch_types=[pltpu.VMEM((NWIN, WIN), jnp.int32),
                       pltpu.VMEM((WIN, F), jnp.float32),
                       pltpu.VMEM_SHARED((N2, F), jnp.float32),
                       pltpu.SemaphoreType.DMA])


_RB = 1000
_GRID = N // _RB


def _dot(a, b):
    return jnp.dot(a, b, preferred_element_type=jnp.float32,
                   precision=lax.Precision.HIGHEST)


def _in_proj_body(x_ref, w_ref, b_ref, o_ref):
    o_ref[...] = _dot(x_ref[...], w_ref[...]) + b_ref[...]


def _in_proj(x, w, b):
    return pl.pallas_call(
        _in_proj_body,
        grid=(_GRID,),
        in_specs=[pl.BlockSpec((_RB, F), lambda i: (i, 0)),
                  pl.BlockSpec((F, F), lambda i: (0, 0)),
                  pl.BlockSpec((1, F), lambda i: (0, 0))],
        out_specs=pl.BlockSpec((_RB, F), lambda i: (i, 0)),
        out_shape=jax.ShapeDtypeStruct((N, F), jnp.float32),
    )(x, w, b.reshape(1, F))


def _mix_body(p_ref, c_ref, h_ref, wl_ref, wr_ref, b_ref, y_ref, s_ref):
    cnt = c_ref[0, :, 0:1] + c_ref[1, :, 0:1]
    inv = 1.0 / jnp.maximum(cnt, 1.0)
    agg = (p_ref[0] + p_ref[1]) * inv
    y = _dot(agg, wl_ref[...]) + _dot(h_ref[...], wr_ref[...]) + b_ref[...]
    y_ref[...] = y
    stats = jnp.concatenate([jnp.sum(y, axis=0, keepdims=True),
                             jnp.sum(y * y, axis=0, keepdims=True)], axis=0)
    i = pl.program_id(0)

    @pl.when(i == 0)
    def _():
        s_ref[...] = stats

    @pl.when(i > 0)
    def _():
        s_ref[...] += stats


def _mix(parts, cnts, h, wl, wr, b):
    return pl.pallas_call(
        _mix_body,
        grid=(_GRID,),
        in_specs=[pl.BlockSpec((NC, _RB, F), lambda i: (0, i, 0)),
                  pl.BlockSpec((NC, _RB, F), lambda i: (0, i, 0)),
                  pl.BlockSpec((_RB, F), lambda i: (i, 0)),
                  pl.BlockSpec((F, F), lambda i: (0, 0)),
                  pl.BlockSpec((F, F), lambda i: (0, 0)),
                  pl.BlockSpec((1, F), lambda i: (0, 0))],
        out_specs=[pl.BlockSpec((_RB, F), lambda i: (i, 0)),
                   pl.BlockSpec((2, F), lambda i: (0, 0))],
        out_shape=[jax.ShapeDtypeStruct((N, F), jnp.float32),
                   jax.ShapeDtypeStruct((2, F), jnp.float32)],
    )(parts, cnts, h, wl, wr, b.reshape(1, F))


def _normed(y, s_ref, w_ref, b_ref, a_ref):
    mean = s_ref[0:1, :] * (1.0 / N)
    msq = s_ref[1:2, :] * (1.0 / N)
    a = a_ref[...]
    var = msq - mean * mean * (2.0 * a - a * a)
    xc = y - a * mean
    t = w_ref[...] * xc / jnp.sqrt(var + 1e-5) + b_ref[...]
    return jnp.maximum(t, 0.1 * t)


def _norm_body(y_ref, s_ref, w_ref, b_ref, a_ref, o_ref):
    o_ref[...] = _normed(y_ref[...], s_ref, w_ref, b_ref, a_ref)


def _norm(y, s, w, b, a):
    return pl.pallas_call(
        _norm_body,
        grid=(_GRID,),
        in_specs=[pl.BlockSpec((_RB, F), lambda i: (i, 0)),
                  pl.BlockSpec((2, F), lambda i: (0, 0)),
                  pl.BlockSpec((1, F), lambda i: (0, 0)),
                  pl.BlockSpec((1, F), lambda i: (0, 0)),
                  pl.BlockSpec((1, F), lambda i: (0, 0))],
        out_specs=pl.BlockSpec((_RB, F), lambda i: (i, 0)),
        out_shape=jax.ShapeDtypeStruct((N, F), jnp.float32),
    )(y, s, w.reshape(1, F), b.reshape(1, F), a.reshape(1, F))


def _norm_out_body(y_ref, s_ref, w_ref, b_ref, a_ref, wo_ref, bo_ref, o_ref):
    t = _normed(y_ref[...], s_ref, w_ref, b_ref, a_ref)
    o_ref[...] = _dot(t, wo_ref[...]) + bo_ref[...]


def _norm_out(y, s, w, b, a, wo, bo):
    return pl.pallas_call(
        _norm_out_body,
        grid=(_GRID,),
        in_specs=[pl.BlockSpec((_RB, F), lambda i: (i, 0)),
                  pl.BlockSpec((2, F), lambda i: (0, 0)),
                  pl.BlockSpec((1, F), lambda i: (0, 0)),
                  pl.BlockSpec((1, F), lambda i: (0, 0)),
                  pl.BlockSpec((1, F), lambda i: (0, 0)),
                  pl.BlockSpec((F, F), lambda i: (0, 0)),
                  pl.BlockSpec((1, F), lambda i: (0, 0))],
        out_specs=pl.BlockSpec((_RB, F), lambda i: (i, 0)),
        out_shape=jax.ShapeDtypeStruct((N, F), jnp.float32),
    )(y, s, w.reshape(1, F), b.reshape(1, F), a.reshape(1, F),
      wo, bo.reshape(1, F))


def kernel(x, edge_index, W_in, b_in,
           W1_l, b1_l, W1_r, gn1_w, gn1_b, gn1_a,
           W2_l, b2_l, W2_r, gn2_w, gn2_b, gn2_a,
           W3_l, b3_l, W3_r, gn3_w, gn3_b, gn3_a,
           W_out, b_out):
    npad = E2 - E
    src = jnp.concatenate(
        [edge_index[0], jnp.zeros((npad,), jnp.int32)])
    dst = jnp.concatenate(
        [edge_index[1], jnp.full((npad,), N, jnp.int32)]).reshape(NW, NWIN, WIN)
    zf = jnp.zeros((N2, F), jnp.float32)
    ones = jnp.ones((WIN, F), jnp.float32)

    _sc_agg = _make_sc_agg()
    _sc_cnt = _make_sc_cnt()

    cnt = _sc_cnt(dst, zf, ones)
    h0 = _in_proj(x, W_in, b_in)
    p1 = _sc_agg(h0, src, dst, zf)
    y1, s1 = _mix(p1, cnt, h0, W1_l, W1_r, b1_l)
    h1 = _norm(y1, s1, gn1_w, gn1_b, gn1_a)
    p2 = _sc_agg(h1, src, dst, zf)
    y2, s2 = _mix(p2, cnt, h1, W2_l, W2_r, b2_l)
    h2 = _norm(y2, s2, gn2_w, gn2_b, gn2_a)
    p3 = _sc_agg(h2, src, dst, zf)
    y3, s3 = _mix(p3, cnt, h2, W3_l, W3_r, b3_l)
    return _norm_out(y3, s3, gn3_w, gn3_b, gn3_a, W_out, b_out)


# speedup vs baseline: 1.0055x; 1.0043x over previous
import jax
import jax.numpy as jnp
from jax import lax
from jax.experimental import pallas as pl
from jax.experimental.pallas import tpu as pltpu
from jax.experimental.pallas import tpu_sc as plsc

N = 10000
E = 320000
F = 128
NC = 2
NS = 16
NW = NC * NS
N2 = N + 32
WIN = 128
NWIN = 80
EPW = WIN * NWIN
E2 = EPW * NW
NBUF = 2
NGRP = NWIN // NBUF
RPS = 624


def _sc_mesh():
    return plsc.VectorSubcoreMesh(core_axis_name="c", subcore_axis_name="s",
                                  num_cores=NC, num_subcores=NS)


def _rows_copy(sid, src, dst, nrows):
    r0 = sid * RPS
    tail = nrows - RPS * NS
    pltpu.sync_copy(src.at[pl.ds(r0, RPS)], dst.at[pl.ds(r0, RPS)])

    @pl.when(sid == 0)
    def _():
        pltpu.sync_copy(src.at[pl.ds(RPS * NS, tail)],
                        dst.at[pl.ds(RPS * NS, tail)])


def _sc_agg_body(h_hbm, src_hbm, dst_hbm, zf_hbm, acc_out,
                 dst_all, i0, i1, r0, r1,
                 is0, is1, g0, g1, s0, s1, acc_sh):
    ibuf = [i0, i1]
    rows = [r0, r1]
    isem = [is0, is1]
    gsem = [g0, g1]
    ssem = [s0, s1]
    cid = lax.axis_index("c")
    sid = lax.axis_index("s")
    wid = cid * NS + sid
    e0 = wid * EPW

    _rows_copy(sid, zf_hbm, acc_sh, N2)
    pltpu.sync_copy(dst_hbm.at[wid], dst_all)

    def idesc(j, b):
        return pltpu.make_async_copy(src_hbm.at[pl.ds(e0 + j * WIN, WIN)],
                                     ibuf[b], isem[b])

    def gdesc(j, b):
        return pltpu.make_async_copy(h_hbm.at[ibuf[b]], rows[b], gsem[b])

    def sdesc(j, b):
        return pltpu.make_async_copy(rows[b], acc_sh.at[dst_all.at[j]],
                                     ssem[b])

    idesc(0, 0).start()
    idesc(1, 1).start()
    plsc.subcore_barrier()
    idesc(0, 0).wait()
    gdesc(0, 0).start()

    def phase(g, b):
        o = 1 - b
        gdesc(g, b).wait()

        @pl.when(g >= 1)
        def _():
            sdesc(g - 1, o).wait()

        sdesc(g, b).start(add=True)

        @pl.when(g + 2 < NWIN)
        def _():
            idesc(g + 2, b).start()

        @pl.when(g + 1 < NWIN)
        def _():
            idesc(g + 1, o).wait()
            gdesc(g + 1, o).start()

    @pl.loop(0, NWIN // 2)
    def _(t):
        phase(2 * t, 0)
        phase(2 * t + 1, 1)

    sdesc(NWIN - 1, 1).wait()
    plsc.subcore_barrier()
    _rows_copy(sid, acc_sh, acc_out.at[cid], N)


def _make_sc_agg():
    return pl.kernel(
        _sc_agg_body,
        out_type=jax.ShapeDtypeStruct((NC, N, F), jnp.float32),
        mesh=_sc_mesh(),
        scratch_types=([pltpu.VMEM((NWIN, WIN), jnp.int32)]
                       + [pltpu.VMEM((WIN,), jnp.int32)] * 2
                       + [pltpu.VMEM((WIN, F), jnp.float32)] * 2
                       + [pltpu.SemaphoreType.DMA] * 6
                       + [pltpu.VMEM_SHARED((N2, F), jnp.float32)]))


def _sc_cnt_body(dst_hbm, zf_hbm, ones_hbm, cnt_out,
                 dst_all, ones_v, cnt_sh, csem):
    cid = lax.axis_index("c")
    sid = lax.axis_index("s")
    wid = cid * NS + sid

    _rows_copy(sid, zf_hbm, cnt_sh, N2)
    pltpu.sync_copy(dst_hbm.at[wid], dst_all)
    pltpu.sync_copy(ones_hbm, ones_v)
    plsc.subcore_barrier()

    @pl.loop(0, NWIN // 4)
    def _(t):
        ds = [pltpu.make_async_copy(ones_v, cnt_sh.at[dst_all.at[4 * t + b]],
                                    csem) for b in range(4)]
        for d in ds:
            d.start(add=True)
        for d in ds:
            d.wait()

    plsc.subcore_barrier()
    _rows_copy(sid, cnt_sh, cnt_out.at[cid], N)


def _make_sc_cnt():
    return pl.kernel(
        _sc_cnt_body,
        out_type=jax.ShapeDtypeStruct((NC, N, F), jnp.float32),
        mesh=_sc_mesh(),
        scratch_types=[pltpu.VMEM((NWIN, WIN), jnp.int32),
                       pltpu.VMEM((WIN, F), jnp.float32),
                       pltpu.VMEM_SHARED((N2, F), jnp.float32),
                       pltpu.SemaphoreType.DMA])


_RB = 1000
_GRID = N // _RB


def _dot(a, b):
    return jnp.dot(a, b, preferred_element_type=jnp.float32,
                   precision=lax.Precision.HIGHEST)


def _in_proj_body(x_ref, w_ref, b_ref, o_ref):
    o_ref[...] = _dot(x_ref[...], w_ref[...]) + b_ref[...]


def _in_proj(x, w, b):
    return pl.pallas_call(
        _in_proj_body,
        grid=(_GRID,),
        in_specs=[pl.BlockSpec((_RB, F), lambda i: (i, 0)),
                  pl.BlockSpec((F, F), lambda i: (0, 0)),
                  pl.BlockSpec((1, F), lambda i: (0, 0))],
        out_specs=pl.BlockSpec((_RB, F), lambda i: (i, 0)),
        out_shape=jax.ShapeDtypeStruct((N, F), jnp.float32),
    )(x, w, b.reshape(1, F))


def _mix_body(p_ref, c_ref, h_ref, wl_ref, wr_ref, b_ref, y_ref, s_ref):
    cnt = c_ref[0, :, 0:1] + c_ref[1, :, 0:1]
    inv = 1.0 / jnp.maximum(cnt, 1.0)
    agg = (p_ref[0] + p_ref[1]) * inv
    y = _dot(agg, wl_ref[...]) + _dot(h_ref[...], wr_ref[...]) + b_ref[...]
    y_ref[...] = y
    stats = jnp.concatenate([jnp.sum(y, axis=0, keepdims=True),
                             jnp.sum(y * y, axis=0, keepdims=True)], axis=0)
    i = pl.program_id(0)

    @pl.when(i == 0)
    def _():
        s_ref[...] = stats

    @pl.when(i > 0)
    def _():
        s_ref[...] += stats


def _mix(parts, cnts, h, wl, wr, b):
    return pl.pallas_call(
        _mix_body,
        grid=(_GRID,),
        in_specs=[pl.BlockSpec((NC, _RB, F), lambda i: (0, i, 0)),
                  pl.BlockSpec((NC, _RB, F), lambda i: (0, i, 0)),
                  pl.BlockSpec((_RB, F), lambda i: (i, 0)),
                  pl.BlockSpec((F, F), lambda i: (0, 0)),
                  pl.BlockSpec((F, F), lambda i: (0, 0)),
                  pl.BlockSpec((1, F), lambda i: (0, 0))],
        out_specs=[pl.BlockSpec((_RB, F), lambda i: (i, 0)),
                   pl.BlockSpec((2, F), lambda i: (0, 0))],
        out_shape=[jax.ShapeDtypeStruct((N, F), jnp.float32),
                   jax.ShapeDtypeStruct((2, F), jnp.float32)],
    )(parts, cnts, h, wl, wr, b.reshape(1, F))


def _normed(y, s_ref, w_ref, b_ref, a_ref):
    mean = s_ref[0:1, :] * (1.0 / N)
    msq = s_ref[1:2, :] * (1.0 / N)
    a = a_ref[...]
    var = msq - mean * mean * (2.0 * a - a * a)
    xc = y - a * mean
    t = w_ref[...] * xc / jnp.sqrt(var + 1e-5) + b_ref[...]
    return jnp.maximum(t, 0.1 * t)


def _norm_body(y_ref, s_ref, w_ref, b_ref, a_ref, o_ref):
    o_ref[...] = _normed(y_ref[...], s_ref, w_ref, b_ref, a_ref)


def _norm(y, s, w, b, a):
    return pl.pallas_call(
        _norm_body,
        grid=(_GRID,),
        in_specs=[pl.BlockSpec((_RB, F), lambda i: (i, 0)),
                  pl.BlockSpec((2, F), lambda i: (0, 0)),
                  pl.BlockSpec((1, F), lambda i: (0, 0)),
                  pl.BlockSpec((1, F), lambda i: (0, 0)),
                  pl.BlockSpec((1, F), lambda i: (0, 0))],
        out_specs=pl.BlockSpec((_RB, F), lambda i: (i, 0)),
        out_shape=jax.ShapeDtypeStruct((N, F), jnp.float32),
    )(y, s, w.reshape(1, F), b.reshape(1, F), a.reshape(1, F))


def _norm_out_body(y_ref, s_ref, w_ref, b_ref, a_ref, wo_ref, bo_ref, o_ref):
    t = _normed(y_ref[...], s_ref, w_ref, b_ref, a_ref)
    o_ref[...] = _dot(t, wo_ref[...]) + bo_ref[...]


def _norm_out(y, s, w, b, a, wo, bo):
    return pl.pallas_call(
        _norm_out_body,
        grid=(_GRID,),
        in_specs=[pl.BlockSpec((_RB, F), lambda i: (i, 0)),
                  pl.BlockSpec((2, F), lambda i: (0, 0)),
                  pl.BlockSpec((1, F), lambda i: (0, 0)),
                  pl.BlockSpec((1, F), lambda i: (0, 0)),
                  pl.BlockSpec((1, F), lambda i: (0, 0)),
                  pl.BlockSpec((F, F), lambda i: (0, 0)),
                  pl.BlockSpec((1, F), lambda i: (0, 0))],
        out_specs=pl.BlockSpec((_RB, F), lambda i: (i, 0)),
        out_shape=jax.ShapeDtypeStruct((N, F), jnp.float32),
    )(y, s, w.reshape(1, F), b.reshape(1, F), a.reshape(1, F),
      wo, bo.reshape(1, F))


def kernel(x, edge_index, W_in, b_in,
           W1_l, b1_l, W1_r, gn1_w, gn1_b, gn1_a,
           W2_l, b2_l, W2_r, gn2_w, gn2_b, gn2_a,
           W3_l, b3_l, W3_r, gn3_w, gn3_b, gn3_a,
           W_out, b_out):
    npad = E2 - E
    src = jnp.concatenate(
        [edge_index[0], jnp.zeros((npad,), jnp.int32)])
    dst = jnp.concatenate(
        [edge_index[1],
         N + (jnp.arange(npad, dtype=jnp.int32) % 32)]).reshape(NW, NWIN, WIN)
    zf = jnp.zeros((N2, F), jnp.float32)
    ones = jnp.ones((WIN, F), jnp.float32)

    _sc_agg = _make_sc_agg()
    _sc_cnt = _make_sc_cnt()

    cnt = _sc_cnt(dst, zf, ones)
    h0 = _in_proj(x, W_in, b_in)
    p1 = _sc_agg(h0, src, dst, zf)
    y1, s1 = _mix(p1, cnt, h0, W1_l, W1_r, b1_l)
    h1 = _norm(y1, s1, gn1_w, gn1_b, gn1_a)
    p2 = _sc_agg(h1, src, dst, zf)
    y2, s2 = _mix(p2, cnt, h1, W2_l, W2_r, b2_l)
    h2 = _norm(y2, s2, gn2_w, gn2_b, gn2_a)
    p3 = _sc_agg(h2, src, dst, zf)
    y3, s3 = _mix(p3, cnt, h2, W3_l, W3_r, b3_l)
    return _norm_out(y3, s3, gn3_w, gn3_b, gn3_a, W_out, b_out)


# speedup vs baseline: 2.7949x; 2.7798x over previous
import jax
import jax.numpy as jnp
from jax import lax
from jax.experimental import pallas as pl
from jax.experimental.pallas import tpu as pltpu
from jax.experimental.pallas import tpu_sc as plsc

N = 10000
E = 320000
F = 128
NC = 2
NS = 16
NW = NC * NS
N2 = N + 32
WIN = 128
NWIN = 80
EPW = WIN * NWIN
E2 = EPW * NW
NBUF = 2
NGRP = NWIN // NBUF
RPS = 624


def _sc_mesh():
    return plsc.VectorSubcoreMesh(core_axis_name="c", subcore_axis_name="s",
                                  num_cores=NC, num_subcores=NS)


def _rows_copy(sid, src, dst, nrows):
    r0 = sid * RPS
    tail = nrows - RPS * NS
    pltpu.sync_copy(src.at[pl.ds(r0, RPS)], dst.at[pl.ds(r0, RPS)])

    @pl.when(sid == 0)
    def _():
        pltpu.sync_copy(src.at[pl.ds(RPS * NS, tail)],
                        dst.at[pl.ds(RPS * NS, tail)])


def _sc_agg_body(h_hbm, src_hbm, dst_hbm, zf_hbm, acc_out,
                 dst_all, i0, i1, r0, r1,
                 is0, is1, g0, g1, s0, s1, acc_sh):
    ibuf = [i0, i1]
    rows = [r0, r1]
    isem = [is0, is1]
    gsem = [g0, g1]
    ssem = [s0, s1]
    cid = lax.axis_index("c")
    sid = lax.axis_index("s")
    wid = cid * NS + sid
    e0 = wid * EPW

    _rows_copy(sid, zf_hbm, acc_sh, N2)
    pltpu.sync_copy(dst_hbm.at[wid], dst_all)

    def idesc(j, b):
        return pltpu.make_async_copy(src_hbm.at[pl.ds(e0 + j * WIN, WIN)],
                                     ibuf[b], isem[b])

    def gdesc(j, b):
        return pltpu.make_async_copy(h_hbm.at[ibuf[b]], rows[b], gsem[b])

    def sdesc(j, b):
        return pltpu.make_async_copy(rows[b], acc_sh.at[dst_all.at[j]],
                                     ssem[b])

    idesc(0, 0).start()
    idesc(1, 1).start()
    plsc.subcore_barrier()
    idesc(0, 0).wait()
    gdesc(0, 0).start()

    def phase(g, b):
        o = 1 - b
        gdesc(g, b).wait()

        @pl.when(g >= 1)
        def _():
            sdesc(g - 1, o).wait()

        sdesc(g, b).start(add=True)

        @pl.when(g + 2 < NWIN)
        def _():
            idesc(g + 2, b).start()

        @pl.when(g + 1 < NWIN)
        def _():
            idesc(g + 1, o).wait()
            gdesc(g + 1, o).start()

    @pl.loop(0, NWIN // 2)
    def _(t):
        phase(2 * t, 0)
        phase(2 * t + 1, 1)

    sdesc(NWIN - 1, 1).wait()
    plsc.subcore_barrier()
    _rows_copy(sid, acc_sh, acc_out.at[cid], N)


def _make_sc_agg():
    return pl.kernel(
        _sc_agg_body,
        out_type=jax.ShapeDtypeStruct((NC, N, F), jnp.float32),
        mesh=_sc_mesh(),
        scratch_types=([pltpu.VMEM((NWIN, WIN), jnp.int32)]
                       + [pltpu.VMEM((WIN,), jnp.int32)] * 2
                       + [pltpu.VMEM((WIN, F), jnp.float32)] * 2
                       + [pltpu.SemaphoreType.DMA] * 6
                       + [pltpu.VMEM_SHARED((N2, F), jnp.float32)]))


def _sc_cnt_body(dst_hbm, zf_hbm, ones_hbm, cnt_out,
                 dst_all, ones_v, cnt_sh, csem):
    cid = lax.axis_index("c")
    sid = lax.axis_index("s")
    wid = cid * NS + sid

    _rows_copy(sid, zf_hbm, cnt_sh, N2)
    pltpu.sync_copy(dst_hbm.at[wid], dst_all)
    pltpu.sync_copy(ones_hbm, ones_v)
    plsc.subcore_barrier()

    @pl.loop(0, NWIN // 4)
    def _(t):
        ds = [pltpu.make_async_copy(ones_v, cnt_sh.at[dst_all.at[4 * t + b]],
                                    csem) for b in range(4)]
        for d in ds:
            d.start(add=True)
        for d in ds:
            d.wait()

    plsc.subcore_barrier()
    _rows_copy(sid, cnt_sh, cnt_out.at[cid], N)


def _make_sc_cnt():
    return pl.kernel(
        _sc_cnt_body,
        out_type=jax.ShapeDtypeStruct((NC, N, F), jnp.float32),
        mesh=_sc_mesh(),
        scratch_types=[pltpu.VMEM((NWIN, WIN), jnp.int32),
                       pltpu.VMEM((WIN, F), jnp.float32),
                       pltpu.VMEM_SHARED((N2, F), jnp.float32),
                       pltpu.SemaphoreType.DMA])


_RB = 1000
_GRID = N // _RB


def _dot(a, b):
    return jnp.dot(a, b, preferred_element_type=jnp.float32,
                   precision=lax.Precision.HIGHEST)


def _in_proj_body(x_ref, w_ref, b_ref, o_ref):
    o_ref[...] = _dot(x_ref[...], w_ref[...]) + b_ref[...]


def _in_proj(x, w, b):
    return pl.pallas_call(
        _in_proj_body,
        grid=(_GRID,),
        in_specs=[pl.BlockSpec((_RB, F), lambda i: (i, 0)),
                  pl.BlockSpec((F, F), lambda i: (0, 0)),
                  pl.BlockSpec((1, F), lambda i: (0, 0))],
        out_specs=pl.BlockSpec((_RB, F), lambda i: (i, 0)),
        out_shape=jax.ShapeDtypeStruct((N, F), jnp.float32),
    )(x, w, b.reshape(1, F))


def _mix_body(p_ref, c_ref, h_ref, wl_ref, wr_ref, b_ref, y_ref, s_ref):
    cnt = c_ref[0, :, 0:1] + c_ref[1, :, 0:1]
    inv = 1.0 / jnp.maximum(cnt, 1.0)
    agg = (p_ref[0] + p_ref[1]) * inv
    y = _dot(agg, wl_ref[...]) + _dot(h_ref[...], wr_ref[...]) + b_ref[...]
    y_ref[...] = y
    stats = jnp.concatenate([jnp.sum(y, axis=0, keepdims=True),
                             jnp.sum(y * y, axis=0, keepdims=True)], axis=0)
    i = pl.program_id(0)

    @pl.when(i == 0)
    def _():
        s_ref[...] = stats

    @pl.when(i > 0)
    def _():
        s_ref[...] += stats


def _mix(parts, cnts, h, wl, wr, b):
    return pl.pallas_call(
        _mix_body,
        grid=(_GRID,),
        in_specs=[pl.BlockSpec((NC, _RB, F), lambda i: (0, i, 0)),
                  pl.BlockSpec((NC, _RB, F), lambda i: (0, i, 0)),
                  pl.BlockSpec((_RB, F), lambda i: (i, 0)),
                  pl.BlockSpec((F, F), lambda i: (0, 0)),
                  pl.BlockSpec((F, F), lambda i: (0, 0)),
                  pl.BlockSpec((1, F), lambda i: (0, 0))],
        out_specs=[pl.BlockSpec((_RB, F), lambda i: (i, 0)),
                   pl.BlockSpec((2, F), lambda i: (0, 0))],
        out_shape=[jax.ShapeDtypeStruct((N, F), jnp.float32),
                   jax.ShapeDtypeStruct((2, F), jnp.float32)],
    )(parts, cnts, h, wl, wr, b.reshape(1, F))


def _normed(y, s_ref, w_ref, b_ref, a_ref):
    mean = s_ref[0:1, :] * (1.0 / N)
    msq = s_ref[1:2, :] * (1.0 / N)
    a = a_ref[...]
    var = msq - mean * mean * (2.0 * a - a * a)
    xc = y - a * mean
    t = w_ref[...] * xc / jnp.sqrt(var + 1e-5) + b_ref[...]
    return jnp.maximum(t, 0.1 * t)


def _norm_body(y_ref, s_ref, w_ref, b_ref, a_ref, o_ref):
    o_ref[...] = _normed(y_ref[...], s_ref, w_ref, b_ref, a_ref)


def _norm(y, s, w, b, a):
    return pl.pallas_call(
        _norm_body,
        grid=(_GRID,),
        in_specs=[pl.BlockSpec((_RB, F), lambda i: (i, 0)),
                  pl.BlockSpec((2, F), lambda i: (0, 0)),
                  pl.BlockSpec((1, F), lambda i: (0, 0)),
                  pl.BlockSpec((1, F), lambda i: (0, 0)),
                  pl.BlockSpec((1, F), lambda i: (0, 0))],
        out_specs=pl.BlockSpec((_RB, F), lambda i: (i, 0)),
        out_shape=jax.ShapeDtypeStruct((N, F), jnp.float32),
    )(y, s, w.reshape(1, F), b.reshape(1, F), a.reshape(1, F))


def _norm_out_body(y_ref, s_ref, w_ref, b_ref, a_ref, wo_ref, bo_ref, o_ref):
    t = _normed(y_ref[...], s_ref, w_ref, b_ref, a_ref)
    o_ref[...] = _dot(t, wo_ref[...]) + bo_ref[...]


def _norm_out(y, s, w, b, a, wo, bo):
    return pl.pallas_call(
        _norm_out_body,
        grid=(_GRID,),
        in_specs=[pl.BlockSpec((_RB, F), lambda i: (i, 0)),
                  pl.BlockSpec((2, F), lambda i: (0, 0)),
                  pl.BlockSpec((1, F), lambda i: (0, 0)),
                  pl.BlockSpec((1, F), lambda i: (0, 0)),
                  pl.BlockSpec((1, F), lambda i: (0, 0)),
                  pl.BlockSpec((F, F), lambda i: (0, 0)),
                  pl.BlockSpec((1, F), lambda i: (0, 0))],
        out_specs=pl.BlockSpec((_RB, F), lambda i: (i, 0)),
        out_shape=jax.ShapeDtypeStruct((N, F), jnp.float32),
    )(y, s, w.reshape(1, F), b.reshape(1, F), a.reshape(1, F),
      wo, bo.reshape(1, F))


def kernel(x, edge_index, W_in, b_in,
           W1_l, b1_l, W1_r, gn1_w, gn1_b, gn1_a,
           W2_l, b2_l, W2_r, gn2_w, gn2_b, gn2_a,
           W3_l, b3_l, W3_r, gn3_w, gn3_b, gn3_a,
           W_out, b_out):
    npad = E2 - E
    pad_iota = jnp.arange(npad, dtype=jnp.int32)
    src = jnp.concatenate([edge_index[0], pad_iota % N])
    dst = jnp.concatenate(
        [edge_index[1], N + (pad_iota % 32)]).reshape(NW, NWIN, WIN)
    zf = jnp.zeros((N2, F), jnp.float32)
    ones = jnp.ones((WIN, F), jnp.float32)

    _sc_agg = _make_sc_agg()
    _sc_cnt = _make_sc_cnt()

    cnt = _sc_cnt(dst, zf, ones)
    h0 = _in_proj(x, W_in, b_in)
    p1 = _sc_agg(h0, src, dst, zf)
    y1, s1 = _mix(p1, cnt, h0, W1_l, W1_r, b1_l)
    h1 = _norm(y1, s1, gn1_w, gn1_b, gn1_a)
    p2 = _sc_agg(h1, src, dst, zf)
    y2, s2 = _mix(p2, cnt, h1, W2_l, W2_r, b2_l)
    h2 = _norm(y2, s2, gn2_w, gn2_b, gn2_a)
    p3 = _sc_agg(h2, src, dst, zf)
    y3, s3 = _mix(p3, cnt, h2, W3_l, W3_r, b3_l)
    return _norm_out(y3, s3, gn3_w, gn3_b, gn3_a, W_out, b_out)


# speedup vs baseline: 2.7973x; 1.0009x over previous
import jax
import jax.numpy as jnp
from jax import lax
from jax.experimental import pallas as pl
from jax.experimental.pallas import tpu as pltpu
from jax.experimental.pallas import tpu_sc as plsc

N = 10000
E = 320000
F = 128
NC = 2
NS = 16
NW = NC * NS
N2 = N + 32
WIN = 128
NWIN = 80
EPW = WIN * NWIN
E2 = EPW * NW
NBUF = 2
NGRP = NWIN // NBUF
RPS = 624


def _sc_mesh():
    return plsc.VectorSubcoreMesh(core_axis_name="c", subcore_axis_name="s",
                                  num_cores=NC, num_subcores=NS)


def _rows_copy(sid, src, dst, nrows):
    r0 = sid * RPS
    tail = nrows - RPS * NS
    pltpu.sync_copy(src.at[pl.ds(r0, RPS)], dst.at[pl.ds(r0, RPS)])

    @pl.when(sid == 0)
    def _():
        pltpu.sync_copy(src.at[pl.ds(RPS * NS, tail)],
                        dst.at[pl.ds(RPS * NS, tail)])


def _sc_agg_body(h_hbm, src_hbm, dst_hbm, zf_hbm, acc_out,
                 i0, i1, i2, j0, j1, j2, r0, r1, r2,
                 is0, is1, is2, js0, js1, js2,
                 g0, g1, g2, s0, s1, s2, acc_sh):
    ibuf = [i0, i1, i2]
    jbuf = [j0, j1, j2]
    rows = [r0, r1, r2]
    isem = [is0, is1, is2]
    jsem = [js0, js1, js2]
    gsem = [g0, g1, g2]
    ssem = [s0, s1, s2]
    cid = lax.axis_index("c")
    sid = lax.axis_index("s")
    wid = cid * NS + sid
    e0 = wid * EPW

    _rows_copy(sid, zf_hbm, acc_sh, N2)

    def idesc(j, b):
        return pltpu.make_async_copy(src_hbm.at[pl.ds(e0 + j * WIN, WIN)],
                                     ibuf[b], isem[b])

    def jdesc(j, b):
        return pltpu.make_async_copy(dst_hbm.at[pl.ds(e0 + j * WIN, WIN)],
                                     jbuf[b], jsem[b])

    def gdesc(j, b):
        return pltpu.make_async_copy(h_hbm.at[ibuf[b]], rows[b], gsem[b])

    def sdesc(j, b):
        return pltpu.make_async_copy(rows[b], acc_sh.at[jbuf[b]], ssem[b])

    for b in range(3):
        idesc(b, b).start()
        jdesc(b, b).start()
    plsc.subcore_barrier()
    idesc(0, 0).wait()
    gdesc(0, 0).start()

    def when(cond, fn):
        if isinstance(cond, bool):
            if cond:
                fn()
        else:
            pl.when(cond)(fn)

    def phase(j, b):
        nb = (b + 1) % 3
        gdesc(j, b).wait()
        when(j >= 2, lambda: sdesc(j - 2, nb).wait())
        when((j >= 2) & (j + 1 < NWIN),
             lambda: jdesc(j + 1, nb).start())
        jdesc(j, b).wait()
        sdesc(j, b).start(add=True)
        when(j + 3 < NWIN, lambda: idesc(j + 3, b).start())
        when(j + 1 < NWIN, lambda: idesc(j + 1, nb).wait())
        when(j + 1 < NWIN, lambda: gdesc(j + 1, nb).start())

    @pl.loop(0, (NWIN - 2) // 3)
    def _(t):
        phase(3 * t, 0)
        phase(3 * t + 1, 1)
        phase(3 * t + 2, 2)

    phase(NWIN - 2, (NWIN - 2) % 3)
    phase(NWIN - 1, (NWIN - 1) % 3)
    sdesc(NWIN - 2, (NWIN - 2) % 3).wait()
    sdesc(NWIN - 1, (NWIN - 1) % 3).wait()
    plsc.subcore_barrier()
    _rows_copy(sid, acc_sh, acc_out.at[cid], N)


def _make_sc_agg():
    return pl.kernel(
        _sc_agg_body,
        out_type=jax.ShapeDtypeStruct((NC, N, F), jnp.float32),
        mesh=_sc_mesh(),
        scratch_types=([pltpu.VMEM((WIN,), jnp.int32)] * 6
                       + [pltpu.VMEM((WIN, F), jnp.float32)] * 3
                       + [pltpu.SemaphoreType.DMA] * 12
                       + [pltpu.VMEM_SHARED((N2, F), jnp.float32)]))


def _sc_cnt_body(dst_hbm, zf_hbm, ones_hbm, cnt_out,
                 dst_all, ones_v, cnt_sh, csem):
    cid = lax.axis_index("c")
    sid = lax.axis_index("s")
    wid = cid * NS + sid

    _rows_copy(sid, zf_hbm, cnt_sh, N2)
    pltpu.sync_copy(dst_hbm.at[wid], dst_all)
    pltpu.sync_copy(ones_hbm, ones_v)
    plsc.subcore_barrier()

    @pl.loop(0, NWIN // 4)
    def _(t):
        ds = [pltpu.make_async_copy(ones_v, cnt_sh.at[dst_all.at[4 * t + b]],
                                    csem) for b in range(4)]
        for d in ds:
            d.start(add=True)
        for d in ds:
            d.wait()

    plsc.subcore_barrier()
    _rows_copy(sid, cnt_sh, cnt_out.at[cid], N)


def _make_sc_cnt():
    return pl.kernel(
        _sc_cnt_body,
        out_type=jax.ShapeDtypeStruct((NC, N, F), jnp.float32),
        mesh=_sc_mesh(),
        scratch_types=[pltpu.VMEM((NWIN, WIN), jnp.int32),
                       pltpu.VMEM((WIN, F), jnp.float32),
                       pltpu.VMEM_SHARED((N2, F), jnp.float32),
                       pltpu.SemaphoreType.DMA])


_RB = 1000
_GRID = N // _RB


def _dot(a, b):
    return jnp.dot(a, b, preferred_element_type=jnp.float32,
                   precision=lax.Precision.HIGHEST)


def _in_proj_body(x_ref, w_ref, b_ref, o_ref):
    o_ref[...] = _dot(x_ref[...], w_ref[...]) + b_ref[...]


def _in_proj(x, w, b):
    return pl.pallas_call(
        _in_proj_body,
        grid=(_GRID,),
        in_specs=[pl.BlockSpec((_RB, F), lambda i: (i, 0)),
                  pl.BlockSpec((F, F), lambda i: (0, 0)),
                  pl.BlockSpec((1, F), lambda i: (0, 0))],
        out_specs=pl.BlockSpec((_RB, F), lambda i: (i, 0)),
        out_shape=jax.ShapeDtypeStruct((N, F), jnp.float32),
    )(x, w, b.reshape(1, F))


def _mix_body(p_ref, c_ref, h_ref, wl_ref, wr_ref, b_ref, y_ref, s_ref):
    cnt = c_ref[0, :, 0:1] + c_ref[1, :, 0:1]
    inv = 1.0 / jnp.maximum(cnt, 1.0)
    agg = (p_ref[0] + p_ref[1]) * inv
    y = _dot(agg, wl_ref[...]) + _dot(h_ref[...], wr_ref[...]) + b_ref[...]
    y_ref[...] = y
    stats = jnp.concatenate([jnp.sum(y, axis=0, keepdims=True),
                             jnp.sum(y * y, axis=0, keepdims=True)], axis=0)
    i = pl.program_id(0)

    @pl.when(i == 0)
    def _():
        s_ref[...] = stats

    @pl.when(i > 0)
    def _():
        s_ref[...] += stats


def _mix(parts, cnts, h, wl, wr, b):
    return pl.pallas_call(
        _mix_body,
        grid=(_GRID,),
        in_specs=[pl.BlockSpec((NC, _RB, F), lambda i: (0, i, 0)),
                  pl.BlockSpec((NC, _RB, F), lambda i: (0, i, 0)),
                  pl.BlockSpec((_RB, F), lambda i: (i, 0)),
                  pl.BlockSpec((F, F), lambda i: (0, 0)),
                  pl.BlockSpec((F, F), lambda i: (0, 0)),
                  pl.BlockSpec((1, F), lambda i: (0, 0))],
        out_specs=[pl.BlockSpec((_RB, F), lambda i: (i, 0)),
                   pl.BlockSpec((2, F), lambda i: (0, 0))],
        out_shape=[jax.ShapeDtypeStruct((N, F), jnp.float32),
                   jax.ShapeDtypeStruct((2, F), jnp.float32)],
    )(parts, cnts, h, wl, wr, b.reshape(1, F))


def _normed(y, s_ref, w_ref, b_ref, a_ref):
    mean = s_ref[0:1, :] * (1.0 / N)
    msq = s_ref[1:2, :] * (1.0 / N)
    a = a_ref[...]
    var = msq - mean * mean * (2.0 * a - a * a)
    xc = y - a * mean
    t = w_ref[...] * xc / jnp.sqrt(var + 1e-5) + b_ref[...]
    return jnp.maximum(t, 0.1 * t)


def _norm_body(y_ref, s_ref, w_ref, b_ref, a_ref, o_ref):
    o_ref[...] = _normed(y_ref[...], s_ref, w_ref, b_ref, a_ref)


def _norm(y, s, w, b, a):
    return pl.pallas_call(
        _norm_body,
        grid=(_GRID,),
        in_specs=[pl.BlockSpec((_RB, F), lambda i: (i, 0)),
                  pl.BlockSpec((2, F), lambda i: (0, 0)),
                  pl.BlockSpec((1, F), lambda i: (0, 0)),
                  pl.BlockSpec((1, F), lambda i: (0, 0)),
                  pl.BlockSpec((1, F), lambda i: (0, 0))],
        out_specs=pl.BlockSpec((_RB, F), lambda i: (i, 0)),
        out_shape=jax.ShapeDtypeStruct((N, F), jnp.float32),
    )(y, s, w.reshape(1, F), b.reshape(1, F), a.reshape(1, F))


def _norm_out_body(y_ref, s_ref, w_ref, b_ref, a_ref, wo_ref, bo_ref, o_ref):
    t = _normed(y_ref[...], s_ref, w_ref, b_ref, a_ref)
    o_ref[...] = _dot(t, wo_ref[...]) + bo_ref[...]


def _norm_out(y, s, w, b, a, wo, bo):
    return pl.pallas_call(
        _norm_out_body,
        grid=(_GRID,),
        in_specs=[pl.BlockSpec((_RB, F), lambda i: (i, 0)),
                  pl.BlockSpec((2, F), lambda i: (0, 0)),
                  pl.BlockSpec((1, F), lambda i: (0, 0)),
                  pl.BlockSpec((1, F), lambda i: (0, 0)),
                  pl.BlockSpec((1, F), lambda i: (0, 0)),
                  pl.BlockSpec((F, F), lambda i: (0, 0)),
                  pl.BlockSpec((1, F), lambda i: (0, 0))],
        out_specs=pl.BlockSpec((_RB, F), lambda i: (i, 0)),
        out_shape=jax.ShapeDtypeStruct((N, F), jnp.float32),
    )(y, s, w.reshape(1, F), b.reshape(1, F), a.reshape(1, F),
      wo, bo.reshape(1, F))


def kernel(x, edge_index, W_in, b_in,
           W1_l, b1_l, W1_r, gn1_w, gn1_b, gn1_a,
           W2_l, b2_l, W2_r, gn2_w, gn2_b, gn2_a,
           W3_l, b3_l, W3_r, gn3_w, gn3_b, gn3_a,
           W_out, b_out):
    npad = E2 - E
    pad_iota = jnp.arange(npad, dtype=jnp.int32)
    src = jnp.concatenate([edge_index[0], pad_iota % N])
    dst = jnp.concatenate([edge_index[1], N + (pad_iota % 32)])
    dst3 = dst.reshape(NW, NWIN, WIN)
    zf = jnp.zeros((N2, F), jnp.float32)
    ones = jnp.ones((WIN, F), jnp.float32)

    _sc_agg = _make_sc_agg()
    _sc_cnt = _make_sc_cnt()

    cnt = _sc_cnt(dst3, zf, ones)
    h0 = _in_proj(x, W_in, b_in)
    p1 = _sc_agg(h0, src, dst, zf)
    y1, s1 = _mix(p1, cnt, h0, W1_l, W1_r, b1_l)
    h1 = _norm(y1, s1, gn1_w, gn1_b, gn1_a)
    p2 = _sc_agg(h1, src, dst, zf)
    y2, s2 = _mix(p2, cnt, h1, W2_l, W2_r, b2_l)
    h2 = _norm(y2, s2, gn2_w, gn2_b, gn2_a)
    p3 = _sc_agg(h2, src, dst, zf)
    y3, s3 = _mix(p3, cnt, h2, W3_l, W3_r, b3_l)
    return _norm_out(y3, s3, gn3_w, gn3_b, gn3_a, W_out, b_out)
